# baseline (speedup 1.0000x reference)
import functools

import jax
import jax.numpy as jnp
from jax.experimental import pallas as pl


def _trunc16(a):
    return jax.lax.bitcast_convert_type(
        jax.lax.bitcast_convert_type(a, jnp.uint32) & jnp.uint32(0xFFFF0000),
        jnp.float32)


def _vq_block(x_ref, cba_ref, cbta_ref, q_ref, idx_ref, loss_ref, *, blk_n):
    b = pl.program_id(0)
    j = pl.program_id(1)

    xb = x_ref[...]
    cba = cba_ref[...]

    xb16 = xb.astype(jnp.bfloat16)
    x_aug = jnp.concatenate(
        [xb16, jnp.ones((8, blk_n), jnp.bfloat16)], axis=0)
    t = jax.lax.dot_general(
        cba, x_aug, (((1,), (0,)), ((), ())),
        preferred_element_type=jnp.float32)

    maxval = jnp.max(t, axis=0)
    onehot = (t == maxval[None, :]).astype(jnp.bfloat16)

    res = jax.lax.dot_general(cbta_ref[...], onehot, (((1,), (0,)), ((), ())),
                              preferred_element_type=jnp.float32)
    q = res[0:32, :] + res[40:72, :]
    q_ref[...] = q
    idx = (2.0 * res[32:33, :] + res[33:34, :]).astype(jnp.int32)
    idx_ref[...] = jnp.broadcast_to(idx, (8, idx.shape[1]))

    part = jnp.sum((q - xb) ** 2).reshape(1, 1)

    @pl.when(jnp.logical_and(b == 0, j == 0))
    def _():
        loss_ref[...] = jnp.zeros_like(loss_ref)

    loss_ref[...] += part


@jax.jit
def kernel(x, codebook):
    B, C, H, W = x.shape
    K = codebook.shape[0]
    N = H * W
    blk_n = 4096

    xr = x.reshape(B, C, N)

    ch = 0.5 * jnp.sum(codebook * codebook, axis=1)
    ch_hi = _trunc16(ch)
    r1 = ch - ch_hi
    ch_mid = _trunc16(r1)
    ch_lo = r1 - ch_mid
    cba = jnp.concatenate(
        [codebook.astype(jnp.bfloat16),
         -ch_hi[:, None].astype(jnp.bfloat16),
         -ch_mid[:, None].astype(jnp.bfloat16),
         -ch_lo[:, None].astype(jnp.bfloat16),
         jnp.zeros((K, 5), jnp.bfloat16)],
        axis=1)

    k_idx = jnp.arange(K, dtype=jnp.float32)
    cbt = codebook.T
    cbt_hi32 = _trunc16(cbt)
    cbt_hi = cbt_hi32.astype(jnp.bfloat16)
    cbt_lo = (cbt - cbt_hi32).astype(jnp.bfloat16)
    cbta = jnp.concatenate(
        [cbt_hi,
         jnp.floor(k_idx / 2.0)[None, :].astype(jnp.bfloat16),
         (k_idx % 2.0)[None, :].astype(jnp.bfloat16),
         jnp.zeros((6, K), jnp.bfloat16),
         cbt_lo],
        axis=0)

    grid = (B, N // blk_n)
    q, idx, loss_sum = pl.pallas_call(
        functools.partial(_vq_block, blk_n=blk_n),
        grid=grid,
        in_specs=[
            pl.BlockSpec((None, C, blk_n), lambda b, j: (b, 0, j)),
            pl.BlockSpec((K, C + 8), lambda b, j: (0, 0)),
            pl.BlockSpec((2 * C + 8, K), lambda b, j: (0, 0)),
        ],
        out_specs=[
            pl.BlockSpec((None, C, blk_n), lambda b, j: (b, 0, j)),
            pl.BlockSpec((None, 8, blk_n), lambda b, j: (b, 0, j)),
            pl.BlockSpec((1, 1), lambda b, j: (0, 0)),
        ],
        out_shape=[
            jax.ShapeDtypeStruct((B, C, N), jnp.float32),
            jax.ShapeDtypeStruct((B, 8, N), jnp.int32),
            jax.ShapeDtypeStruct((1, 1), jnp.float32),
        ],
    )(xr, cba, cbta)

    quantized = q.reshape(B, C, H, W)
    indices = idx[:, 0, :].reshape(B, H, W)
    commit_loss = (loss_sum[0, 0] / (B * N * C)).reshape(())
    return quantized, indices, commit_loss


# speedup vs baseline: 1.4586x; 1.4586x over previous
import functools

import jax
import jax.numpy as jnp
from jax.experimental import pallas as pl


def _trunc16(a):
    return jax.lax.bitcast_convert_type(
        jax.lax.bitcast_convert_type(a, jnp.uint32) & jnp.uint32(0xFFFF0000),
        jnp.float32)


def _vq_block(x_ref, cba_ref, cbta_ref, q_ref, idx_ref, loss_ref, *, n_tok):
    b = pl.program_id(0)

    xb3 = x_ref[...]
    C, H, W = xb3.shape
    cba = cba_ref[...]

    xb16 = xb3.astype(jnp.bfloat16).reshape(C, n_tok)
    x_aug = jnp.concatenate(
        [xb16, jnp.ones((8, n_tok), jnp.bfloat16)], axis=0)
    t = jax.lax.dot_general(
        cba, x_aug, (((1,), (0,)), ((), ())),
        preferred_element_type=jnp.float32)

    maxval = jnp.max(t, axis=0)
    onehot = (t == maxval[None, :]).astype(jnp.bfloat16)

    res = jax.lax.dot_general(cbta_ref[...], onehot, (((1,), (0,)), ((), ())),
                              preferred_element_type=jnp.float32)
    q3 = (res[0:32, :] + res[40:72, :]).reshape(C, H, W)
    q_ref[...] = q3
    idx = (2.0 * res[32:33, :] + res[33:34, :]).astype(jnp.int32)
    idx_ref[...] = idx.reshape(H, W)

    part = jnp.sum((q3 - xb3) ** 2).reshape(1, 1)

    @pl.when(b == 0)
    def _():
        loss_ref[...] = jnp.zeros_like(loss_ref)

    loss_ref[...] += part


@jax.jit
def kernel(x, codebook):
    B, C, H, W = x.shape
    K = codebook.shape[0]
    N = H * W

    ch = 0.5 * jnp.sum(codebook * codebook, axis=1)
    ch_hi = _trunc16(ch)
    r1 = ch - ch_hi
    ch_mid = _trunc16(r1)
    ch_lo = r1 - ch_mid
    cba = jnp.concatenate(
        [codebook.astype(jnp.bfloat16),
         -ch_hi[:, None].astype(jnp.bfloat16),
         -ch_mid[:, None].astype(jnp.bfloat16),
         -ch_lo[:, None].astype(jnp.bfloat16),
         jnp.zeros((K, 5), jnp.bfloat16)],
        axis=1)

    k_idx = jnp.arange(K, dtype=jnp.float32)
    cbt = codebook.T
    cbt_hi32 = _trunc16(cbt)
    cbt_hi = cbt_hi32.astype(jnp.bfloat16)
    cbt_lo = (cbt - cbt_hi32).astype(jnp.bfloat16)
    cbta = jnp.concatenate(
        [cbt_hi,
         jnp.floor(k_idx / 2.0)[None, :].astype(jnp.bfloat16),
         (k_idx % 2.0)[None, :].astype(jnp.bfloat16),
         jnp.zeros((6, K), jnp.bfloat16),
         cbt_lo],
        axis=0)

    grid = (B,)
    quantized, indices, loss_sum = pl.pallas_call(
        functools.partial(_vq_block, n_tok=N),
        grid=grid,
        in_specs=[
            pl.BlockSpec((None, C, H, W), lambda b: (b, 0, 0, 0)),
            pl.BlockSpec((K, C + 8), lambda b: (0, 0)),
            pl.BlockSpec((2 * C + 8, K), lambda b: (0, 0)),
        ],
        out_specs=[
            pl.BlockSpec((None, C, H, W), lambda b: (b, 0, 0, 0)),
            pl.BlockSpec((None, H, W), lambda b: (b, 0, 0)),
            pl.BlockSpec((1, 1), lambda b: (0, 0)),
        ],
        out_shape=[
            jax.ShapeDtypeStruct((B, C, H, W), jnp.float32),
            jax.ShapeDtypeStruct((B, H, W), jnp.int32),
            jax.ShapeDtypeStruct((1, 1), jnp.float32),
        ],
    )(x, cba, cbta)

    commit_loss = (loss_sum[0, 0] / (B * N * C)).reshape(())
    return quantized, indices, commit_loss


# speedup vs baseline: 1.5617x; 1.0707x over previous
import functools

import jax
import jax.numpy as jnp
from jax.experimental import pallas as pl


def _trunc16(a):
    return jax.lax.bitcast_convert_type(
        jax.lax.bitcast_convert_type(a, jnp.uint32) & jnp.uint32(0xFFFF0000),
        jnp.float32)


def _vq_block(x_ref, cba_ref, cbta_ref, q_ref, idx_ref, loss_ref, *, n_tok):
    b = pl.program_id(0)
    cba = cba_ref[...]
    cbta = cbta_ref[...]

    part = jnp.zeros((1, 1), jnp.float32)
    for i in range(x_ref.shape[0]):
        xb3 = x_ref[i]
        C, H, W = xb3.shape

        xb16 = xb3.astype(jnp.bfloat16).reshape(C, n_tok)
        x_aug = jnp.concatenate(
            [xb16, jnp.ones((8, n_tok), jnp.bfloat16)], axis=0)
        t = jax.lax.dot_general(
            cba, x_aug, (((1,), (0,)), ((), ())),
            preferred_element_type=jnp.float32)

        maxval = jnp.max(t, axis=0)
        onehot = (t == maxval[None, :]).astype(jnp.bfloat16)

        res = jax.lax.dot_general(cbta, onehot, (((1,), (0,)), ((), ())),
                                  preferred_element_type=jnp.float32)
        q3 = (res[0:32, :] + res[40:72, :]).reshape(C, H, W)
        q_ref[i] = q3
        idx = (2.0 * res[32:33, :] + res[33:34, :]).astype(jnp.int32)
        idx_ref[i] = idx.reshape(H, W)

        part = part + jnp.sum((q3 - xb3) ** 2).reshape(1, 1)

    @pl.when(b == 0)
    def _():
        loss_ref[...] = jnp.zeros_like(loss_ref)

    loss_ref[...] += part


@jax.jit
def kernel(x, codebook):
    B, C, H, W = x.shape
    K = codebook.shape[0]
    N = H * W

    ch = 0.5 * jnp.sum(codebook * codebook, axis=1)
    ch_hi = _trunc16(ch)
    r1 = ch - ch_hi
    ch_mid = _trunc16(r1)
    ch_lo = r1 - ch_mid
    cba = jnp.concatenate(
        [codebook.astype(jnp.bfloat16),
         -ch_hi[:, None].astype(jnp.bfloat16),
         -ch_mid[:, None].astype(jnp.bfloat16),
         -ch_lo[:, None].astype(jnp.bfloat16),
         jnp.zeros((K, 5), jnp.bfloat16)],
        axis=1)

    k_idx = jnp.arange(K, dtype=jnp.float32)
    cbt = codebook.T
    cbt_hi32 = _trunc16(cbt)
    cbt_hi = cbt_hi32.astype(jnp.bfloat16)
    cbt_lo = (cbt - cbt_hi32).astype(jnp.bfloat16)
    cbta = jnp.concatenate(
        [cbt_hi,
         jnp.floor(k_idx / 2.0)[None, :].astype(jnp.bfloat16),
         (k_idx % 2.0)[None, :].astype(jnp.bfloat16),
         jnp.zeros((6, K), jnp.bfloat16),
         cbt_lo],
        axis=0)

    bb = 2
    grid = (B // bb,)
    quantized, indices, loss_sum = pl.pallas_call(
        functools.partial(_vq_block, n_tok=N),
        grid=grid,
        in_specs=[
            pl.BlockSpec((bb, C, H, W), lambda b: (b, 0, 0, 0)),
            pl.BlockSpec((K, C + 8), lambda b: (0, 0)),
            pl.BlockSpec((2 * C + 8, K), lambda b: (0, 0)),
        ],
        out_specs=[
            pl.BlockSpec((bb, C, H, W), lambda b: (b, 0, 0, 0)),
            pl.BlockSpec((bb, H, W), lambda b: (b, 0, 0)),
            pl.BlockSpec((1, 1), lambda b: (0, 0)),
        ],
        out_shape=[
            jax.ShapeDtypeStruct((B, C, H, W), jnp.float32),
            jax.ShapeDtypeStruct((B, H, W), jnp.int32),
            jax.ShapeDtypeStruct((1, 1), jnp.float32),
        ],
    )(x, cba, cbta)

    commit_loss = (loss_sum[0, 0] / (B * N * C)).reshape(())
    return quantized, indices, commit_loss


# speedup vs baseline: 1.6199x; 1.0373x over previous
import functools

import jax
import jax.numpy as jnp
from jax.experimental import pallas as pl


def _trunc16(a):
    return jax.lax.bitcast_convert_type(
        jax.lax.bitcast_convert_type(a, jnp.uint32) & jnp.uint32(0xFFFF0000),
        jnp.float32)


def _vq_block(x_ref, cba_ref, cbta_ref, q_ref, idx_ref, loss_ref, *, n_tok):
    b = pl.program_id(0)
    cba = cba_ref[...]
    cbta = cbta_ref[...]

    part = jnp.zeros((1, 1), jnp.float32)
    for i in range(x_ref.shape[0]):
        xb3 = x_ref[i]
        C, H, W = xb3.shape

        xb16 = xb3.astype(jnp.bfloat16).reshape(C, n_tok)
        x_aug = jnp.concatenate(
            [xb16, jnp.ones((8, n_tok), jnp.bfloat16)], axis=0)
        t = jax.lax.dot_general(
            cba, x_aug, (((1,), (0,)), ((), ())),
            preferred_element_type=jnp.float32)

        maxval = jnp.max(t, axis=0)
        onehot = (t == maxval[None, :]).astype(jnp.bfloat16)

        res = jax.lax.dot_general(cbta, onehot, (((1,), (0,)), ((), ())),
                                  preferred_element_type=jnp.float32)
        q3 = (res[0:32, :] + res[40:72, :]).reshape(C, H, W)
        q_ref[i] = q3
        idx = (2.0 * res[32:33, :] + res[33:34, :]).astype(jnp.int32)
        idx_ref[i] = idx.reshape(H, W)

        part = part + jnp.sum((q3 - xb3) ** 2).reshape(1, 1)

    @pl.when(b == 0)
    def _():
        loss_ref[...] = jnp.zeros_like(loss_ref)

    loss_ref[...] += part


@jax.jit
def kernel(x, codebook):
    B, C, H, W = x.shape
    K = codebook.shape[0]
    N = H * W

    ch = 0.5 * jnp.sum(codebook * codebook, axis=1)
    ch_hi = _trunc16(ch)
    r1 = ch - ch_hi
    ch_mid = _trunc16(r1)
    ch_lo = r1 - ch_mid
    cba = jnp.concatenate(
        [codebook.astype(jnp.bfloat16),
         -ch_hi[:, None].astype(jnp.bfloat16),
         -ch_mid[:, None].astype(jnp.bfloat16),
         -ch_lo[:, None].astype(jnp.bfloat16),
         jnp.zeros((K, 5), jnp.bfloat16)],
        axis=1)

    k_idx = jnp.arange(K, dtype=jnp.float32)
    cbt = codebook.T
    cbt_hi32 = _trunc16(cbt)
    cbt_hi = cbt_hi32.astype(jnp.bfloat16)
    cbt_lo = (cbt - cbt_hi32).astype(jnp.bfloat16)
    cbta = jnp.concatenate(
        [cbt_hi,
         jnp.floor(k_idx / 2.0)[None, :].astype(jnp.bfloat16),
         (k_idx % 2.0)[None, :].astype(jnp.bfloat16),
         jnp.zeros((6, K), jnp.bfloat16),
         cbt_lo],
        axis=0)

    bb = 4
    grid = (B // bb,)
    quantized, indices, loss_sum = pl.pallas_call(
        functools.partial(_vq_block, n_tok=N),
        grid=grid,
        in_specs=[
            pl.BlockSpec((bb, C, H, W), lambda b: (b, 0, 0, 0)),
            pl.BlockSpec((K, C + 8), lambda b: (0, 0)),
            pl.BlockSpec((2 * C + 8, K), lambda b: (0, 0)),
        ],
        out_specs=[
            pl.BlockSpec((bb, C, H, W), lambda b: (b, 0, 0, 0)),
            pl.BlockSpec((bb, H, W), lambda b: (b, 0, 0)),
            pl.BlockSpec((1, 1), lambda b: (0, 0)),
        ],
        out_shape=[
            jax.ShapeDtypeStruct((B, C, H, W), jnp.float32),
            jax.ShapeDtypeStruct((B, H, W), jnp.int32),
            jax.ShapeDtypeStruct((1, 1), jnp.float32),
        ],
    )(x, cba, cbta)

    commit_loss = (loss_sum[0, 0] / (B * N * C)).reshape(())
    return quantized, indices, commit_loss
